# Initial kernel scaffold; baseline (speedup 1.0000x reference)
import functools

import jax
import jax.numpy as jnp
from jax import lax
from jax.experimental import pallas as pl
from jax.experimental.pallas import tpu as pltpu
from jax.experimental.pallas import tpu_sc as plsc

N = 10000
E = 160000
D_IN = 256
D_H = 256
D_OUT = 128

NP = 10240
EP = 163840
PAD_E = EP - E
ROWS_PER_TILE = NP // 16
CHUNK = 128

F32 = jnp.float32
I32 = jnp.int32


def _mesh():
    return plsc.VectorSubcoreMesh(
        core_axis_name="c", subcore_axis_name="s", num_cores=2, num_subcores=16
    )


def _zero_rows(buf, nrows, width):
    zero16 = jnp.zeros((16,), F32)

    def zrow(r, carry):
        for l in range(width // 16):
            buf[r, pl.ds(l * 16, 16)] = zero16
        return carry

    lax.fori_loop(0, nrows, zrow, 0)


def _rsqrt16(d):
    xi = plsc.bitcast(d, I32)
    yi = jnp.int32(0x5F3759DF) - lax.shift_right_logical(xi, 1)
    y = plsc.bitcast(yi, F32)
    for _ in range(4):
        y = y * (1.5 - 0.5 * d * y * y)
    return y


def _make_sc_deg():
    @functools.partial(
        pl.kernel,
        out_type=jax.ShapeDtypeStruct((NP // 16, 16), F32),
        mesh=_mesh(),
        scratch_types=[
            pltpu.VMEM((EP // 16 // CHUNK, CHUNK), I32),
            pltpu.VMEM((NP // 16, 16), F32),
            pltpu.VMEM((ROWS_PER_TILE // 16, 16), F32),
            pltpu.VMEM((ROWS_PER_TILE // 16, 16), F32),
            pltpu.VMEM((NP // 16 // CHUNK, CHUNK), I32),
            pltpu.VMEM_SHARED((NP // 16, 16), F32),
        ],
    )
    def k(dst16, rowidx, norm_out, dstbuf, acc, degbuf, normbuf, rowbuf, sh_deg):
        c = lax.axis_index("c")
        s = lax.axis_index("s")

        @pl.when(c == 0)
        def _():
            pltpu.sync_copy(dst16.at[s], dstbuf)
            pltpu.sync_copy(rowidx, rowbuf)
            _zero_rows(acc, NP // 16, 16)
            nrows = ROWS_PER_TILE // 16
            pltpu.sync_copy(acc.at[pl.ds(0, nrows)], sh_deg.at[pl.ds(s * nrows, nrows)])
            plsc.subcore_barrier()

            ones16 = jnp.ones((16,), F32)

            def step(j, carry):
                for l in range(CHUNK // 16):
                    idx = dstbuf[j, pl.ds(l * 16, 16)]
                    r = lax.shift_right_logical(idx, 4)
                    cc = lax.bitwise_and(idx, jnp.int32(15))
                    plsc.addupdate_scatter(acc, [r, cc], ones16)
                return carry

            lax.fori_loop(0, EP // 16 // CHUNK, step, 0)

            for j in range(NP // 16 // CHUNK):
                pltpu.sync_copy(
                    acc.at[pl.ds(j * CHUNK, CHUNK)],
                    sh_deg.at[rowbuf.at[j]],
                    add=True,
                )
            plsc.subcore_barrier()

            pltpu.sync_copy(sh_deg.at[pl.ds(s * nrows, nrows)], degbuf)

            def nrow(r, carry):
                d = degbuf[r]
                nv = jnp.where(d > 0.5, _rsqrt16(d), jnp.zeros((16,), F32))
                normbuf[r] = nv
                return carry

            lax.fori_loop(0, nrows, nrow, 0)
            pltpu.sync_copy(normbuf, norm_out.at[pl.ds(s * nrows, nrows)])

    return k


def _make_sc_agg():
    @functools.partial(
        pl.kernel,
        out_type=jax.ShapeDtypeStruct((2, NP, 128), F32),
        mesh=_mesh(),
        scratch_types=[
            pltpu.VMEM((EP // 16 // CHUNK, CHUNK), I32),
            pltpu.VMEM((EP // 16 // CHUNK, CHUNK), I32),
            pltpu.VMEM((CHUNK, 128), F32),
            pltpu.VMEM_SHARED((NP, 128), F32),
        ],
    )
    def k(ys0, ys1, src16, dst16, out, srcbuf, dstbuf, msgs, acc):
        c = lax.axis_index("c")
        s = lax.axis_index("s")
        pltpu.sync_copy(src16.at[s], srcbuf)
        pltpu.sync_copy(dst16.at[s], dstbuf)
        _zero_rows(msgs, CHUNK, 128)
        base = s * ROWS_PER_TILE
        for j in range(ROWS_PER_TILE // CHUNK):
            pltpu.sync_copy(msgs, acc.at[pl.ds(base + j * CHUNK, CHUNK)])
        plsc.subcore_barrier()

        def run(ys_ref):
            def step(j, carry):
                pltpu.sync_copy(ys_ref.at[srcbuf.at[j]], msgs)
                pltpu.sync_copy(msgs, acc.at[dstbuf.at[j]], add=True)
                return carry

            lax.fori_loop(0, EP // 16 // CHUNK, step, 0)

        @pl.when(c == 0)
        def _():
            run(ys0)

        @pl.when(c == 1)
        def _():
            run(ys1)

        plsc.subcore_barrier()
        for j in range(ROWS_PER_TILE // CHUNK):
            pltpu.sync_copy(
                acc.at[pl.ds(base + j * CHUNK, CHUNK)],
                out.at[c, pl.ds(base + j * CHUNK, CHUNK)],
            )

    return k


def _make_sc_agg3():
    @functools.partial(
        pl.kernel,
        out_type=jax.ShapeDtypeStruct((2, NP, 128), F32),
        mesh=_mesh(),
        scratch_types=[
            pltpu.VMEM((EP // 32 // CHUNK, CHUNK), I32),
            pltpu.VMEM((EP // 32 // CHUNK, CHUNK), I32),
            pltpu.VMEM((CHUNK, 128), F32),
            pltpu.VMEM_SHARED((NP, 128), F32),
        ],
    )
    def k(ys3, src32, dst32, out, srcbuf, dstbuf, msgs, acc):
        c = lax.axis_index("c")
        s = lax.axis_index("s")
        w = c * 16 + s
        pltpu.sync_copy(src32.at[w], srcbuf)
        pltpu.sync_copy(dst32.at[w], dstbuf)
        _zero_rows(msgs, CHUNK, 128)
        base = s * ROWS_PER_TILE
        for j in range(ROWS_PER_TILE // CHUNK):
            pltpu.sync_copy(msgs, acc.at[pl.ds(base + j * CHUNK, CHUNK)])
        plsc.subcore_barrier()

        def step(j, carry):
            pltpu.sync_copy(ys3.at[srcbuf.at[j]], msgs)
            pltpu.sync_copy(msgs, acc.at[dstbuf.at[j]], add=True)
            return carry

        lax.fori_loop(0, EP // 32 // CHUNK, step, 0)
        plsc.subcore_barrier()
        for j in range(ROWS_PER_TILE // CHUNK):
            pltpu.sync_copy(
                acc.at[pl.ds(base + j * CHUNK, CHUNK)],
                out.at[c, pl.ds(base + j * CHUNK, CHUNK)],
            )

    return k


_GRID = NP // 512


def _tc1(featp, W1, normb):
    def body(f_ref, w_ref, nb_ref, o0_ref, o1_ref):
        y = jnp.dot(f_ref[...], w_ref[...], preferred_element_type=F32)
        ys = y * nb_ref[...]
        o0_ref[...] = ys[:, :128]
        o1_ref[...] = ys[:, 128:]

    return pl.pallas_call(
        body,
        grid=(_GRID,),
        in_specs=[
            pl.BlockSpec((512, D_IN), lambda i: (i, 0)),
            pl.BlockSpec((D_IN, D_H), lambda i: (0, 0)),
            pl.BlockSpec((512, 256), lambda i: (i, 0)),
        ],
        out_specs=[
            pl.BlockSpec((512, 128), lambda i: (i, 0)),
            pl.BlockSpec((512, 128), lambda i: (i, 0)),
        ],
        out_shape=[
            jax.ShapeDtypeStruct((NP, 128), F32),
            jax.ShapeDtypeStruct((NP, 128), F32),
        ],
    )(featp, W1, normb)


def _tc_mid(agg, W, b, normb):
    def body(a_ref, w_ref, b_ref, nb_ref, o0_ref, o1_ref):
        nb = nb_ref[...]
        h = jnp.concatenate([a_ref[0], a_ref[1]], axis=1)
        h = jnp.maximum(h * nb + b_ref[...], 0.0)
        y = jnp.dot(h, w_ref[...], preferred_element_type=F32)
        ys = y * nb
        o0_ref[...] = ys[:, :128]
        o1_ref[...] = ys[:, 128:]

    return pl.pallas_call(
        body,
        grid=(_GRID,),
        in_specs=[
            pl.BlockSpec((2, 512, 128), lambda i: (0, i, 0)),
            pl.BlockSpec((D_H, D_H), lambda i: (0, 0)),
            pl.BlockSpec((1, D_H), lambda i: (0, 0)),
            pl.BlockSpec((512, 256), lambda i: (i, 0)),
        ],
        out_specs=[
            pl.BlockSpec((512, 128), lambda i: (i, 0)),
            pl.BlockSpec((512, 128), lambda i: (i, 0)),
        ],
        out_shape=[
            jax.ShapeDtypeStruct((NP, 128), F32),
            jax.ShapeDtypeStruct((NP, 128), F32),
        ],
    )(agg, W, b, normb)


def _tc3(agg, W, b, normb):
    def body(a_ref, w_ref, b_ref, nb_ref, o_ref):
        nb = nb_ref[...]
        h = jnp.concatenate([a_ref[0], a_ref[1]], axis=1)
        h = jnp.maximum(h * nb + b_ref[...], 0.0)
        y = jnp.dot(h, w_ref[...], preferred_element_type=F32)
        o_ref[...] = y * nb[:, :128]

    return pl.pallas_call(
        body,
        grid=(_GRID,),
        in_specs=[
            pl.BlockSpec((2, 512, 128), lambda i: (0, i, 0)),
            pl.BlockSpec((D_H, D_OUT), lambda i: (0, 0)),
            pl.BlockSpec((1, D_H), lambda i: (0, 0)),
            pl.BlockSpec((512, 256), lambda i: (i, 0)),
        ],
        out_specs=pl.BlockSpec((512, 128), lambda i: (i, 0)),
        out_shape=jax.ShapeDtypeStruct((NP, 128), F32),
    )(agg, W, b, normb)


def _tc_final(agg3, b3, normb):
    def body(a_ref, b_ref, nb_ref, o_ref):
        agg = a_ref[0] + a_ref[1]
        o_ref[...] = agg * nb_ref[...][:, :128] + b_ref[...]

    return pl.pallas_call(
        body,
        grid=(_GRID,),
        in_specs=[
            pl.BlockSpec((2, 512, 128), lambda i: (0, i, 0)),
            pl.BlockSpec((1, D_OUT), lambda i: (0, 0)),
            pl.BlockSpec((512, 256), lambda i: (i, 0)),
        ],
        out_specs=pl.BlockSpec((512, 128), lambda i: (i, 0)),
        out_shape=jax.ShapeDtypeStruct((NP, 128), F32),
    )(agg3, b3, normb)


def kernel(features, edge_index, W1, b1, W2, b2, W3, b3):
    ei = edge_index.astype(I32)
    src = jnp.concatenate([ei[0], jnp.zeros((PAD_E,), I32)])
    dst = jnp.concatenate([ei[1], jnp.full((PAD_E,), NP - 1, I32)])
    src16 = src.reshape(16, EP // 16 // CHUNK, CHUNK)
    dst16 = dst.reshape(16, EP // 16 // CHUNK, CHUNK)
    src32 = src.reshape(32, EP // 32 // CHUNK, CHUNK)
    dst32 = dst.reshape(32, EP // 32 // CHUNK, CHUNK)
    rowidx = jnp.arange(NP // 16, dtype=I32).reshape(NP // 16 // CHUNK, CHUNK)

    featp = jnp.pad(features.astype(F32), ((0, NP - N), (0, 0)))
    W1 = W1.astype(F32)
    W2 = W2.astype(F32)
    W3 = W3.astype(F32)

    norm = _make_sc_deg()(dst16, rowidx)
    normb = jnp.broadcast_to(norm.reshape(NP, 1), (NP, 256))

    ys0, ys1 = _tc1(featp, W1, normb)
    agg1 = _make_sc_agg()(ys0, ys1, src16, dst16)
    ys0, ys1 = _tc_mid(agg1, W2, b1.reshape(1, D_H).astype(F32), normb)
    agg2 = _make_sc_agg()(ys0, ys1, src16, dst16)
    ys3 = _tc3(agg2, W3, b2.reshape(1, D_H).astype(F32), normb)
    agg3 = _make_sc_agg3()(ys3, src32, dst32)
    out = _tc_final(agg3, b3.reshape(1, D_OUT).astype(F32), normb)
    return out[:N]


# speedup vs baseline: 66.7739x; 66.7739x over previous
import functools

import numpy as np

import jax
import jax.numpy as jnp
from jax import lax
from jax.experimental import pallas as pl
from jax.experimental.pallas import tpu as pltpu
from jax.experimental.pallas import tpu_sc as plsc

N = 10000
E = 160000
D_IN = 256
D_H = 256
D_OUT = 128

NP = 10240
EP = 163840
PAD_E = EP - E
ROWS_PER_TILE = NP // 16
CHUNK = 128
NCH16 = EP // 16 // CHUNK
NCH32 = EP // 32 // CHUNK

F32 = jnp.float32
I32 = jnp.int32


def _mesh():
    return plsc.VectorSubcoreMesh(
        core_axis_name="c", subcore_axis_name="s", num_cores=2, num_subcores=16
    )


def _zero_rows(buf, nrows, width):
    zero16 = jnp.zeros((16,), F32)

    def zrow(r, carry):
        for l in range(width // 16):
            buf[r, pl.ds(l * 16, 16)] = zero16
        return carry

    lax.fori_loop(jnp.int32(0), jnp.int32(nrows), zrow, jnp.int32(0))


def _make_sc_count():
    @functools.partial(
        pl.kernel,
        out_type=jax.ShapeDtypeStruct((2 * NP, 128), F32),
        mesh=_mesh(),
        scratch_types=[
            pltpu.VMEM((NCH16, CHUNK), I32),
            pltpu.VMEM((CHUNK, 128), F32),
            pltpu.VMEM_SHARED((NP, 128), F32),
        ],
    )
    def k(dst2, out, dstbuf, onesbuf, acc):
        c = lax.axis_index("c")
        s = lax.axis_index("s")
        pltpu.sync_copy(dst2.at[pl.ds(s * NCH16, NCH16)], dstbuf)
        _zero_rows(onesbuf, CHUNK, 128)
        base = s * ROWS_PER_TILE
        for j in range(ROWS_PER_TILE // CHUNK):
            pltpu.sync_copy(onesbuf, acc.at[pl.ds(base + j * CHUNK, CHUNK)])

        one16 = jnp.ones((16,), F32)

        def orow(r, carry):
            for l in range(128 // 16):
                onesbuf[r, pl.ds(l * 16, 16)] = one16
            return carry

        lax.fori_loop(jnp.int32(0), jnp.int32(CHUNK), orow, jnp.int32(0))
        plsc.subcore_barrier()

        def step(j, carry):
            pltpu.sync_copy(onesbuf, acc.at[dstbuf.at[j]], add=True)
            return carry

        lax.fori_loop(jnp.int32(0), jnp.int32(NCH16), step, jnp.int32(0))
        plsc.subcore_barrier()

        obase = c * NP + base
        for j in range(ROWS_PER_TILE // CHUNK):
            pltpu.sync_copy(
                acc.at[pl.ds(base + j * CHUNK, CHUNK)],
                out.at[pl.ds(obase + j * CHUNK, CHUNK)],
            )

    return k


def _make_sc_agg():
    @functools.partial(
        pl.kernel,
        out_type=jax.ShapeDtypeStruct((2 * NP, 128), F32),
        mesh=_mesh(),
        scratch_types=[
            pltpu.VMEM((NCH16, CHUNK), I32),
            pltpu.VMEM((NCH16, CHUNK), I32),
            pltpu.VMEM((CHUNK, 128), F32),
            pltpu.VMEM_SHARED((NP, 128), F32),
        ],
    )
    def k(ys0, ys1, src2, dst2, out, srcbuf, dstbuf, msgs, acc):
        c = lax.axis_index("c")
        s = lax.axis_index("s")
        pltpu.sync_copy(src2.at[pl.ds(s * NCH16, NCH16)], srcbuf)
        pltpu.sync_copy(dst2.at[pl.ds(s * NCH16, NCH16)], dstbuf)
        _zero_rows(msgs, CHUNK, 128)
        base = s * ROWS_PER_TILE
        for j in range(ROWS_PER_TILE // CHUNK):
            pltpu.sync_copy(msgs, acc.at[pl.ds(base + j * CHUNK, CHUNK)])
        plsc.subcore_barrier()

        def run(ys_ref):
            def step(j, carry):
                pltpu.sync_copy(ys_ref.at[srcbuf.at[j]], msgs)
                pltpu.sync_copy(msgs, acc.at[dstbuf.at[j]], add=True)
                return carry

            lax.fori_loop(jnp.int32(0), jnp.int32(NCH16), step, jnp.int32(0))

        @pl.when(c == 0)
        def _():
            run(ys0)

        @pl.when(c == 1)
        def _():
            run(ys1)

        plsc.subcore_barrier()
        obase = c * NP + base
        for j in range(ROWS_PER_TILE // CHUNK):
            pltpu.sync_copy(
                acc.at[pl.ds(base + j * CHUNK, CHUNK)],
                out.at[pl.ds(obase + j * CHUNK, CHUNK)],
            )

    return k


def _make_sc_agg3():
    @functools.partial(
        pl.kernel,
        out_type=jax.ShapeDtypeStruct((2 * NP, 128), F32),
        mesh=_mesh(),
        scratch_types=[
            pltpu.VMEM((NCH32, CHUNK), I32),
            pltpu.VMEM((NCH32, CHUNK), I32),
            pltpu.VMEM((CHUNK, 128), F32),
            pltpu.VMEM_SHARED((NP, 128), F32),
        ],
    )
    def k(ys3, src2, dst2, out, srcbuf, dstbuf, msgs, acc):
        c = lax.axis_index("c")
        s = lax.axis_index("s")
        w = c * 16 + s
        pltpu.sync_copy(src2.at[pl.ds(w * NCH32, NCH32)], srcbuf)
        pltpu.sync_copy(dst2.at[pl.ds(w * NCH32, NCH32)], dstbuf)
        _zero_rows(msgs, CHUNK, 128)
        base = s * ROWS_PER_TILE
        for j in range(ROWS_PER_TILE // CHUNK):
            pltpu.sync_copy(msgs, acc.at[pl.ds(base + j * CHUNK, CHUNK)])
        plsc.subcore_barrier()

        def step(j, carry):
            pltpu.sync_copy(ys3.at[srcbuf.at[j]], msgs)
            pltpu.sync_copy(msgs, acc.at[dstbuf.at[j]], add=True)
            return carry

        lax.fori_loop(jnp.int32(0), jnp.int32(NCH32), step, jnp.int32(0))
        plsc.subcore_barrier()
        obase = c * NP + base
        for j in range(ROWS_PER_TILE // CHUNK):
            pltpu.sync_copy(
                acc.at[pl.ds(base + j * CHUNK, CHUNK)],
                out.at[pl.ds(obase + j * CHUNK, CHUNK)],
            )

    return k


_GRID = NP // 512
_I0 = np.int32(0)


def _norm_tile(d_ref):
    d = d_ref[...]
    return jnp.where(d > 0.5, lax.rsqrt(d), 0.0).astype(F32)


def _tc1(featp, W1, degb):
    def body(f_ref, w_ref, d_ref, o0_ref, o1_ref):
        nb = _norm_tile(d_ref)
        y = jnp.dot(f_ref[...], w_ref[...], preferred_element_type=F32)
        o0_ref[...] = y[:, :128] * nb
        o1_ref[...] = y[:, 128:] * nb

    return pl.pallas_call(
        body,
        grid=(_GRID,),
        in_specs=[
            pl.BlockSpec((512, D_IN), lambda i: (i, _I0)),
            pl.BlockSpec((D_IN, D_H), lambda i: (_I0, _I0)),
            pl.BlockSpec((512, 128), lambda i: (i, _I0)),
        ],
        out_specs=[
            pl.BlockSpec((512, 128), lambda i: (i, _I0)),
            pl.BlockSpec((512, 128), lambda i: (i, _I0)),
        ],
        out_shape=[
            jax.ShapeDtypeStruct((NP, 128), F32),
            jax.ShapeDtypeStruct((NP, 128), F32),
        ],
    )(featp, W1, degb)


def _tc_mid(agg, W, b, degb):
    def body(a_ref, w_ref, b_ref, d_ref, o0_ref, o1_ref):
        nb = _norm_tile(d_ref)
        nb2 = jnp.concatenate([nb, nb], axis=1)
        h = jnp.concatenate([a_ref[0], a_ref[1]], axis=1)
        h = jnp.maximum(h * nb2 + b_ref[...], 0.0)
        y = jnp.dot(h, w_ref[...], preferred_element_type=F32)
        o0_ref[...] = y[:, :128] * nb
        o1_ref[...] = y[:, 128:] * nb

    return pl.pallas_call(
        body,
        grid=(_GRID,),
        in_specs=[
            pl.BlockSpec((2, 512, 128), lambda i: (_I0, i, _I0)),
            pl.BlockSpec((D_H, D_H), lambda i: (_I0, _I0)),
            pl.BlockSpec((1, D_H), lambda i: (_I0, _I0)),
            pl.BlockSpec((512, 128), lambda i: (i, _I0)),
        ],
        out_specs=[
            pl.BlockSpec((512, 128), lambda i: (i, _I0)),
            pl.BlockSpec((512, 128), lambda i: (i, _I0)),
        ],
        out_shape=[
            jax.ShapeDtypeStruct((NP, 128), F32),
            jax.ShapeDtypeStruct((NP, 128), F32),
        ],
    )(agg, W, b, degb)


def _tc3(agg, W, b, degb):
    def body(a_ref, w_ref, b_ref, d_ref, o_ref):
        nb = _norm_tile(d_ref)
        nb2 = jnp.concatenate([nb, nb], axis=1)
        h = jnp.concatenate([a_ref[0], a_ref[1]], axis=1)
        h = jnp.maximum(h * nb2 + b_ref[...], 0.0)
        y = jnp.dot(h, w_ref[...], preferred_element_type=F32)
        o_ref[...] = y * nb

    return pl.pallas_call(
        body,
        grid=(_GRID,),
        in_specs=[
            pl.BlockSpec((2, 512, 128), lambda i: (_I0, i, _I0)),
            pl.BlockSpec((D_H, D_OUT), lambda i: (_I0, _I0)),
            pl.BlockSpec((1, D_H), lambda i: (_I0, _I0)),
            pl.BlockSpec((512, 128), lambda i: (i, _I0)),
        ],
        out_specs=pl.BlockSpec((512, 128), lambda i: (i, _I0)),
        out_shape=jax.ShapeDtypeStruct((NP, 128), F32),
    )(agg, W, b, degb)


def _tc_final(agg3, b3, degb):
    def body(a_ref, b_ref, d_ref, o_ref):
        nb = _norm_tile(d_ref)
        o_ref[...] = (a_ref[0] + a_ref[1]) * nb + b_ref[...]

    return pl.pallas_call(
        body,
        grid=(_GRID,),
        in_specs=[
            pl.BlockSpec((2, 512, 128), lambda i: (_I0, i, _I0)),
            pl.BlockSpec((1, D_OUT), lambda i: (_I0, _I0)),
            pl.BlockSpec((512, 128), lambda i: (i, _I0)),
        ],
        out_specs=pl.BlockSpec((512, 128), lambda i: (i, _I0)),
        out_shape=jax.ShapeDtypeStruct((NP, 128), F32),
    )(agg3, b3, degb)


def kernel(features, edge_index, W1, b1, W2, b2, W3, b3):
    ei = edge_index.astype(I32)
    src = jnp.concatenate([ei[0], jnp.zeros((PAD_E,), I32)])
    dst = jnp.concatenate([ei[1], jnp.full((PAD_E,), NP - 1, I32)])
    src2 = src.reshape(EP // CHUNK, CHUNK)
    dst2 = dst.reshape(EP // CHUNK, CHUNK)

    featp = jnp.pad(features.astype(F32), ((0, NP - N), (0, 0)))
    W1 = W1.astype(F32)
    W2 = W2.astype(F32)
    W3 = W3.astype(F32)

    degb = _make_sc_count()(dst2)[:NP]

    ys0, ys1 = _tc1(featp, W1, degb)
    agg1 = _make_sc_agg()(ys0, ys1, src2, dst2).reshape(2, NP, 128)
    ys0, ys1 = _tc_mid(agg1, W2, b1.reshape(1, D_H).astype(F32), degb)
    agg2 = _make_sc_agg()(ys0, ys1, src2, dst2).reshape(2, NP, 128)
    ys3 = _tc3(agg2, W3, b2.reshape(1, D_H).astype(F32), degb)
    agg3 = _make_sc_agg3()(ys3, src2, dst2).reshape(2, NP, 128)
    out = _tc_final(agg3, b3.reshape(1, D_OUT).astype(F32), degb)
    return out[:N].astype(jnp.float64)
